# baseline (speedup 1.0000x reference)
import functools

import jax
import jax.numpy as jnp
from jax import lax
from jax.experimental import pallas as pl
from jax.experimental.pallas import tpu as pltpu
from jax.experimental.pallas import tpu_sc as plsc

B, N, D = 64, 1024, 768
NC, NS, L = 2, 16, 16
NW = NC * NS
CHUNK = (N // NW) * D
U = 8
VECS = CHUNK // L
NBUF = 4
ROUNDS = B // NBUF

_mesh = plsc.VectorSubcoreMesh(core_axis_name="c", subcore_axis_name="s")


@functools.partial(
    pl.kernel,
    mesh=_mesh,
    out_type=jax.ShapeDtypeStruct((B * N * D,), jnp.float32),
    scratch_types=(
        [pltpu.VMEM((CHUNK,), jnp.float32) for _ in range(NBUF)]
        + [pltpu.VMEM((CHUNK,), jnp.float32)]
        + [pltpu.SemaphoreType.DMA for _ in range(2 * NBUF)]
    ),
)
def _sc_pos_add(patch_hbm, pos_hbm, out_hbm, *refs):
    ios = list(refs[:NBUF])
    pos_v = refs[NBUF]
    sin = list(refs[NBUF + 1 : NBUF + 1 + NBUF])
    sout = list(refs[NBUF + 1 + NBUF :])

    wid = lax.axis_index("s") * NC + lax.axis_index("c")
    pos_off = wid * CHUNK
    pltpu.sync_copy(pos_hbm.at[pl.ds(pos_off, CHUNK)], pos_v)

    for j in range(NBUF):
        off = j * (N * D) + pos_off
        pltpu.async_copy(patch_hbm.at[pl.ds(off, CHUNK)], ios[j], sin[j])

    def round_body(g, carry):
        b0 = g * NBUF
        for j in range(NBUF):
            off = (b0 + j) * (N * D) + pos_off
            pltpu.make_async_copy(patch_hbm.at[pl.ds(off, CHUNK)], ios[j], sin[j]).wait()

            def add_body(i, c, io=ios[j]):
                s = pl.multiple_of(i * (L * U), L)
                for u in range(U):
                    sl = pl.ds(s + u * L, L)
                    plsc.addupdate(io.at[sl], pos_v[sl])
                return c

            lax.fori_loop(0, VECS // U, add_body, 0)
            pltpu.async_copy(ios[j], out_hbm.at[pl.ds(off, CHUNK)], sout[j])

        @pl.when(g + 1 < ROUNDS)
        def _():
            for j in range(NBUF):
                off_n = (b0 + NBUF + j) * (N * D) + pos_off
                pltpu.make_async_copy(ios[j], out_hbm.at[pl.ds(pos_off, CHUNK)], sout[j]).wait()
                pltpu.async_copy(patch_hbm.at[pl.ds(off_n, CHUNK)], ios[j], sin[j])

        return carry

    lax.fori_loop(0, ROUNDS, round_body, 0)

    for j in range(NBUF):
        pltpu.make_async_copy(ios[j], out_hbm.at[pl.ds(pos_off, CHUNK)], sout[j]).wait()


def kernel(patch, pos_table):
    out = _sc_pos_add(patch.reshape(-1), pos_table.reshape(-1))
    return out.reshape(B, N, D)


# speedup vs baseline: 2.6334x; 2.6334x over previous
import functools

import jax
import jax.numpy as jnp
from jax import lax
from jax.experimental import pallas as pl
from jax.experimental.pallas import tpu as pltpu
from jax.experimental.pallas import tpu_sc as plsc

B, N, D = 64, 1024, 768
NC, NS, L = 2, 16, 16
NW = NC * NS
RPW = N // NW
U = 8
NBUF = 4
ROUNDS = B // NBUF

_mesh = plsc.VectorSubcoreMesh(core_axis_name="c", subcore_axis_name="s")


@functools.partial(
    pl.kernel,
    mesh=_mesh,
    out_type=jax.ShapeDtypeStruct((B, N, D), jnp.float32),
    scratch_types=(
        [pltpu.VMEM((RPW, D), jnp.float32) for _ in range(NBUF + 1)]
        + [pltpu.SemaphoreType.DMA for _ in range(2 * NBUF)]
    ),
)
def _sc_pos_add(patch_hbm, pos_hbm, out_hbm, *refs):
    ios = list(refs[:NBUF])
    pos_v = refs[NBUF]
    sin = list(refs[NBUF + 1 : NBUF + 1 + NBUF])
    sout = list(refs[NBUF + 1 + NBUF :])

    wid = lax.axis_index("s") * NC + lax.axis_index("c")
    n0 = wid * RPW
    pltpu.sync_copy(pos_hbm.at[pl.ds(n0, RPW)], pos_v)

    for j in range(NBUF):
        pltpu.async_copy(patch_hbm.at[j, pl.ds(n0, RPW)], ios[j], sin[j])

    def add_pos(io):
        def row_body(r, c):
            for k in range(D // (L * U)):
                for u in range(U):
                    sl = pl.ds(k * L * U + u * L, L)
                    plsc.addupdate(io.at[r, sl], pos_v[r, sl])
            return c

        lax.fori_loop(0, RPW, row_body, 0)

    def round_body(g, carry):
        b0 = g * NBUF
        for j in range(NBUF):
            b = b0 + j
            pltpu.make_async_copy(patch_hbm.at[b, pl.ds(n0, RPW)], ios[j], sin[j]).wait()
            add_pos(ios[j])
            pltpu.async_copy(ios[j], out_hbm.at[b, pl.ds(n0, RPW)], sout[j])

        @pl.when(g + 1 < ROUNDS)
        def _():
            for j in range(NBUF):
                pltpu.make_async_copy(ios[j], out_hbm.at[b0, pl.ds(n0, RPW)], sout[j]).wait()
                pltpu.async_copy(patch_hbm.at[b0 + NBUF + j, pl.ds(n0, RPW)], ios[j], sin[j])

        return carry

    lax.fori_loop(0, ROUNDS, round_body, 0)

    for j in range(NBUF):
        pltpu.make_async_copy(ios[j], out_hbm.at[0, pl.ds(n0, RPW)], sout[j]).wait()


def kernel(patch, pos_table):
    return _sc_pos_add(patch, pos_table)
